# Initial kernel scaffold; baseline (speedup 1.0000x reference)
import jax
import jax.numpy as jnp
from jax.experimental import pallas as pl


def kernel(c_0, e, t, W_in, b_in, We1, be1, We2, be2, Wn1, bn1, Wn2, bn2, W_out, b_out):
    raise NotImplementedError("write your pallas kernel here")


# speedup vs baseline: 3.8564x; 3.8564x over previous
import functools

import jax
import jax.numpy as jnp
import numpy as np
from jax import lax
from jax.experimental import pallas as pl
from jax.experimental.pallas import tpu as pltpu
from jax.experimental.pallas import tpu_sc as plsc

NUM_STEPS = 100
K = 20
NL = 4
H = 23
HP = 32
L = 100000
E = 3200000
CH = 128
NCHUNK = E // CH
NW = 32
SH = L // 2
SBUF_ROWS = 50048
BLKL = 2000
BLKE = 8000


def _schedule_np():
    T = NUM_STEPS
    s = 0.01
    tt = np.arange(0, T + 1, dtype=np.float64)
    f_t = np.cos((np.pi / 2.0) * ((tt / T) + s) / (1 + s)) ** 2
    alpha_bars = f_t / f_t[0]
    betas = 1.0 - (alpha_bars[1:] / alpha_bars[:-1])
    betas = np.concatenate([[0.0], betas]).clip(max=0.999)
    alphas = 1.0 - betas
    return (jnp.asarray(betas, jnp.float32), jnp.asarray(alphas, jnp.float32),
            jnp.asarray(alpha_bars, jnp.float32))


_BETAS, _ALPHAS, _ALPHA_BARS = _schedule_np()

@functools.lru_cache(maxsize=None)
def _sc_mesh():
    return plsc.VectorSubcoreMesh(core_axis_name="c", subcore_axis_name="s",
                                  num_cores=2, num_subcores=16)


def _silu(x):
    return x / (1.0 + jnp.exp(-x))


def _gather_body(a_hbm, b_hbm, src_hbm, dst_hbm, m1_hbm,
                 sidx, didx, bufa, bufb, sema, semb):
    c = lax.axis_index("c")
    s = lax.axis_index("s")
    w = s * 2 + c
    n = (NCHUNK - 1 - w) // NW + 1

    def body(j, carry):
        base = (w + j * NW) * CH
        pltpu.sync_copy(src_hbm.at[pl.ds(base, CH)], sidx)
        pltpu.sync_copy(dst_hbm.at[pl.ds(base, CH)], didx)
        cpa = pltpu.async_copy(a_hbm.at[sidx], bufa, sema)
        cpb = pltpu.async_copy(b_hbm.at[didx], bufb, semb)
        cpa.wait()
        cpb.wait()

        def row(r, carry2):
            bufa[r, pl.ds(0, 16)] = bufa[r, pl.ds(0, 16)] + bufb[r, pl.ds(0, 16)]
            bufa[r, pl.ds(16, 16)] = (bufa[r, pl.ds(16, 16)]
                                      + bufb[r, pl.ds(16, 16)])
            return carry2

        lax.fori_loop(0, CH, row, 0, unroll=8)
        pltpu.sync_copy(bufa, m1_hbm.at[pl.ds(base, CH)])
        return carry

    lax.fori_loop(0, n, body, 0)


@functools.partial(jax.jit, donate_argnums=())
def _sc_gather(a_tab, b_tab, src, dst):
    return pl.kernel(
        _gather_body,
        out_type=jax.ShapeDtypeStruct((E, HP), jnp.float32),
        mesh=_sc_mesh(),
        scratch_types=[
            pltpu.VMEM((CH,), jnp.int32),
            pltpu.VMEM((CH,), jnp.int32),
            pltpu.VMEM((CH, HP), jnp.float32),
            pltpu.VMEM((CH, HP), jnp.float32),
            pltpu.SemaphoreType.DMA,
            pltpu.SemaphoreType.DMA,
        ],
        compiler_params=pltpu.CompilerParams(use_tc_tiling_on_sc=False),
        name="egnn_edge_gather",
    )(a_tab, b_tab, src, dst)


def _scatter_body(m2_hbm, dst_hbm, agg_hbm, didx, lidx, bufm, sbuf):
    c = lax.axis_index("c")
    s = lax.axis_index("s")
    node_base = c * SH

    def zrow(r, carry):
        bufm[r, pl.ds(0, 16)] = jnp.zeros((16,), jnp.float32)
        bufm[r, pl.ds(16, 16)] = jnp.zeros((16,), jnp.float32)
        return carry

    lax.fori_loop(0, CH, zrow, 0, unroll=8)
    nz = (SBUF_ROWS // CH - 1 - s) // 16 + 1

    def zchunk(j, carry):
        k = s + j * 16
        pltpu.sync_copy(bufm, sbuf.at[pl.ds(k * CH, CH)])
        return carry

    lax.fori_loop(0, nz, zchunk, 0)
    plsc.subcore_barrier()

    n = (NCHUNK - 1 - s) // 16 + 1

    def body(j, carry):
        base = (s + j * 16) * CH
        pltpu.sync_copy(dst_hbm.at[pl.ds(base, CH)], didx)
        pltpu.sync_copy(m2_hbm.at[pl.ds(base, CH)], bufm)
        for i in range(CH // 16):
            v = didx[pl.ds(i * 16, 16)] - node_base
            ok = (v >= 0) & (v < SH)
            lidx[pl.ds(i * 16, 16)] = jnp.where(ok, v, SH)
        pltpu.sync_copy(bufm, sbuf.at[lidx], add=True)
        return carry

    lax.fori_loop(0, n, body, 0)
    plsc.subcore_barrier()

    nco = (L // 2000 - 1 - s) // 16 + 1

    def cochunk(j, carry):
        k = s + j * 16
        pltpu.sync_copy(sbuf.at[pl.ds(k * 1000, 1000)],
                        agg_hbm.at[pl.ds(node_base + k * 1000, 1000)])
        return carry

    lax.fori_loop(0, nco, cochunk, 0)


@jax.jit
def _sc_scatter(m2, dst):
    return pl.kernel(
        _scatter_body,
        out_type=jax.ShapeDtypeStruct((L, HP), jnp.float32),
        mesh=_sc_mesh(),
        scratch_types=[
            pltpu.VMEM((CH,), jnp.int32),
            pltpu.VMEM((CH,), jnp.int32),
            pltpu.VMEM((CH, HP), jnp.float32),
            pltpu.VMEM_SHARED((SBUF_ROWS, HP), jnp.float32),
        ],
        compiler_params=pltpu.CompilerParams(use_tc_tiling_on_sc=False),
        name="egnn_edge_scatter",
    )(m2, dst)


def _stage_a_body(x0_ref, winp_ref, binp_ref, wa_ref, wb_ref, be1_ref,
                  h_ref, a_ref, b_ref):
    h = (jnp.dot(x0_ref[...], winp_ref[...], preferred_element_type=jnp.float32)
         + binp_ref[...])
    h_ref[...] = h
    a_ref[...] = jnp.dot(h, wa_ref[...], preferred_element_type=jnp.float32)
    b_ref[...] = (jnp.dot(h, wb_ref[...], preferred_element_type=jnp.float32)
                  + be1_ref[...])


def _node_body(h_ref, agg_ref, wn1a_ref, wn1b_ref, bn1_ref, wn2_ref, bn2_ref,
               wa_ref, ba_ref, wb_ref, bb_ref, hn_ref, a_ref, b_ref):
    h = h_ref[...]
    u = _silu(jnp.dot(h, wn1a_ref[...], preferred_element_type=jnp.float32)
              + jnp.dot(agg_ref[...], wn1b_ref[...],
                        preferred_element_type=jnp.float32)
              + bn1_ref[...])
    hn = h + jnp.dot(u, wn2_ref[...], preferred_element_type=jnp.float32) + bn2_ref[...]
    hn_ref[...] = hn
    a_ref[...] = (jnp.dot(hn, wa_ref[...], preferred_element_type=jnp.float32)
                  + ba_ref[...])
    b_ref[...] = (jnp.dot(hn, wb_ref[...], preferred_element_type=jnp.float32)
                  + bb_ref[...])


def _edge_body(m1_ref, w_ref, b_ref, m2_ref):
    x = _silu(m1_ref[...])
    m2_ref[...] = _silu(jnp.dot(x, w_ref[...], preferred_element_type=jnp.float32)
                        + b_ref[...])


def _loss_body(lg_ref, c0_ref, ct_ref, par_ref, out_ref):
    i = pl.program_id(0)
    lg = lg_ref[...]
    mx = jnp.max(lg, axis=-1, keepdims=True)
    p = jnp.exp(lg - mx)
    c_den = p / jnp.sum(p, axis=-1, keepdims=True)
    alpha = par_ref[0, 0]
    ab = par_ref[0, 1]
    c0 = c0_ref[...]
    ct = ct_ref[...]
    kmask = lax.broadcasted_iota(jnp.int32, (BLKL, HP), 1) < K
    f1 = alpha * ct + (1.0 - alpha) / K
    th_t = jnp.where(kmask, f1 * (ab * c0 + (1.0 - ab) / K), 0.0)
    post_true = th_t / (jnp.sum(th_t, axis=-1, keepdims=True) + 1e-8)
    th_p = jnp.where(kmask, f1 * (ab * c_den + (1.0 - ab) / K), 0.0)
    post_pred = th_p / (jnp.sum(th_p, axis=-1, keepdims=True) + 1e-8)
    lpp = jnp.log(post_pred + 1e-8)
    kl = jnp.where(post_true > 0, post_true * jnp.log(post_true), 0.0) \
        - post_true * lpp
    kl = jnp.where(kmask, kl, 0.0)
    blk = jnp.sum(kl, axis=(0, 1), keepdims=True)[:1, :1]

    @pl.when(i == 0)
    def _():
        out_ref[...] = jnp.zeros_like(out_ref)

    out_ref[...] += blk


def _full_spec():
    return pl.BlockSpec((HP, HP), lambda i: (0, 0))


def _bias_spec():
    return pl.BlockSpec((1, HP), lambda i: (0, 0))


def _tc_stage_a(x0, winp, binp, wa, wb, be1):
    grid = L // BLKL
    blk = pl.BlockSpec((BLKL, HP), lambda i: (i, 0))
    out = jax.ShapeDtypeStruct((L, HP), jnp.float32)
    return pl.pallas_call(
        _stage_a_body,
        grid=(grid,),
        in_specs=[blk, _full_spec(), _bias_spec(), _full_spec(), _full_spec(),
                  _bias_spec()],
        out_specs=[blk, blk, blk],
        out_shape=[out, out, out],
    )(x0, winp, binp, wa, wb, be1)


def _tc_node(h, agg, wn1a, wn1b, bn1, wn2, bn2, wa, ba, wb, bb):
    grid = L // BLKL
    blk = pl.BlockSpec((BLKL, HP), lambda i: (i, 0))
    out = jax.ShapeDtypeStruct((L, HP), jnp.float32)
    return pl.pallas_call(
        _node_body,
        grid=(grid,),
        in_specs=[blk, blk, _full_spec(), _full_spec(), _bias_spec(),
                  _full_spec(), _bias_spec(), _full_spec(), _bias_spec(),
                  _full_spec(), _bias_spec()],
        out_specs=[blk, blk, blk],
        out_shape=[out, out, out],
    )(h, agg, wn1a, wn1b, bn1, wn2, bn2, wa, ba, wb, bb)


def _tc_edge(m1, w, b):
    grid = E // BLKE
    blk = pl.BlockSpec((BLKE, HP), lambda i: (i, 0))
    return pl.pallas_call(
        _edge_body,
        grid=(grid,),
        in_specs=[blk, _full_spec(), _bias_spec()],
        out_specs=blk,
        out_shape=jax.ShapeDtypeStruct((E, HP), jnp.float32),
    )(m1, w, b)


def _tc_loss(logits, c0p, ctp, par):
    grid = L // BLKL
    blk = pl.BlockSpec((BLKL, HP), lambda i: (i, 0))
    acc = pl.pallas_call(
        _loss_body,
        grid=(grid,),
        in_specs=[blk, blk, blk, pl.BlockSpec((1, 128), lambda i: (0, 0))],
        out_specs=pl.BlockSpec((1, 1), lambda i: (0, 0)),
        out_shape=jax.ShapeDtypeStruct((1, 1), jnp.float32),
        compiler_params=pltpu.CompilerParams(
            dimension_semantics=("arbitrary",)),
    )(logits, c0p, ctp, par)
    return acc[0, 0]


def _pad2(w):
    return jnp.zeros((HP, HP), jnp.float32).at[:w.shape[0], :w.shape[1]].set(w)


def _padb(b):
    return jnp.zeros((1, HP), jnp.float32).at[0, :b.shape[0]].set(b)


def kernel(c_0, e, t, W_in, b_in, We1, be1, We2, be2, Wn1, bn1, Wn2, bn2,
           W_out, b_out):
    src = e[0, 0]
    dst = e[1, 0]

    key = jax.random.key(42)
    k1, k2 = jax.random.split(key)
    s_0 = jax.random.categorical(k1, jnp.log(c_0 + 1e-8), axis=-1)
    c0_oh = jax.nn.one_hot(s_0, K, dtype=jnp.float32)
    ab = _ALPHA_BARS[t][:, None, None]
    c_noisy = ab * c0_oh + (1.0 - ab) / K
    s_noisy = jax.random.categorical(k2, jnp.log(c_noisy + 1e-8), axis=-1)
    beta = _BETAS[t][0]
    alpha = _ALPHAS[t][0]
    ab0 = ab[0, 0, 0]

    x0 = jnp.zeros((L, HP), jnp.float32)
    x0 = x0.at[:, :K].set(c_noisy[0])
    x0 = x0.at[:, K].set(beta)
    x0 = x0.at[:, K + 1].set(jnp.sin(beta))
    x0 = x0.at[:, K + 2].set(jnp.cos(beta))
    c0p = jnp.zeros((L, HP), jnp.float32).at[:, :K].set(c0_oh[0])
    ctp = jnp.zeros((L, HP), jnp.float32).at[:, :K].set(
        jax.nn.one_hot(s_noisy[0], K, dtype=jnp.float32))

    winp = _pad2(W_in)
    binp = _padb(b_in)
    we1a = [_pad2(We1[l, :H]) for l in range(NL)]
    we1b = [_pad2(We1[l, H:]) for l in range(NL)]
    be1p = [_padb(be1[l]) for l in range(NL)]
    we2p = [_pad2(We2[l]) for l in range(NL)]
    be2p = [_padb(be2[l]) for l in range(NL)]
    wn1a = [_pad2(Wn1[l, :H]) for l in range(NL)]
    wn1b = [_pad2(Wn1[l, H:]) for l in range(NL)]
    bn1p = [_padb(bn1[l]) for l in range(NL)]
    wn2p = [_pad2(Wn2[l]) for l in range(NL)]
    bn2p = [_padb(bn2[l]) for l in range(NL)]
    w_outp = _pad2(W_out)
    b_outp = jnp.full((1, HP), -1e30, jnp.float32).at[0, :K].set(b_out)
    zero_w = jnp.zeros((HP, HP), jnp.float32)
    zero_b = jnp.zeros((1, HP), jnp.float32)

    h, a_tab, b_tab = _tc_stage_a(x0, winp, binp, we1a[0], we1b[0], be1p[0])
    for l in range(NL):
        m1 = _sc_gather(a_tab, b_tab, src, dst)
        m2 = _tc_edge(m1, we2p[l], be2p[l])
        agg = _sc_scatter(m2, dst)
        if l < NL - 1:
            wa, ba, wb, bb = we1a[l + 1], zero_b, we1b[l + 1], be1p[l + 1]
        else:
            wa, ba, wb, bb = w_outp, b_outp, zero_w, zero_b
        h, a_tab, b_tab = _tc_node(h, agg, wn1a[l], wn1b[l], bn1p[l],
                                   wn2p[l], bn2p[l], wa, ba, wb, bb)

    par = jnp.zeros((1, 128), jnp.float32).at[0, 0].set(alpha).at[0, 1].set(ab0)
    total = _tc_loss(a_tab, c0p, ctp, par)
    return total / (L + 1e-8)
